# baseline (speedup 1.0000x reference)
import math

import jax
import jax.numpy as jnp
from jax import lax
from jax.experimental import pallas as pl
from jax.experimental.pallas import tpu as pltpu
from jax.experimental.pallas import tpu_sc as plsc

B = 64
S = 2048
D = 128
V = 9
L = 16
NC = 2
NS = 16
NW = NC * NS
SPW = S // NW
ROWS = V * SPW
SCALE = math.sqrt(D)
G = 2
NCHUNK = B // G


def _pe_body(x_hbm, tab_hbm, pos_hbm, out_hbm,
             cmb_sp, tabs_v, pos_v, idx_v, gbuf0, gbuf1,
             gs0, gs1, ws0, ws1, xs):
    cid = lax.axis_index("c")
    sid = lax.axis_index("s")
    wid = sid * NC + cid
    s0 = wid * SPW

    xh = [pltpu.async_copy(x_hbm.at[pl.ds(b * S + s0, SPW)], idx_v.at[b], xs)
          for b in range(B)]
    pltpu.sync_copy(tab_hbm, tabs_v)
    pltpu.sync_copy(pos_hbm.at[pl.ds(s0, SPW)], pos_v)

    for v in range(V):
        for d0 in range(0, D, L):
            sl = pl.ds(d0, L)
            tabs_v[v, sl] = tabs_v[v, sl] * SCALE

    pieces = (gbuf0.at[0], gbuf0.at[1], gbuf1.at[0], gbuf1.at[1])
    psems = (gs0, gs1, ws0, ws1)
    sh = [None] * V
    for v in range(V):
        p = v % 4
        if v >= 4:
            sh[v - 4].wait()
        piece = pieces[p]
        tv = [tabs_v[v, pl.ds(d0, L)] for d0 in range(0, D, L)]

        def build_j(j2, carry, piece=piece, tv=tv):
            for u in range(2):
                j = j2 * 2 + u
                for k, d0 in enumerate(range(0, D, L)):
                    sl = pl.ds(d0, L)
                    piece[j, sl] = pos_v[j, sl] + tv[k]
            return carry

        lax.fori_loop(0, SPW // 2, build_j, 0)
        sh[v] = pltpu.async_copy(
            piece, cmb_sp.at[pl.ds(sid * ROWS + v * SPW, SPW)], psems[p])

    for h in xh:
        h.wait()
    base = sid * ROWS
    jvecs = [lax.iota(jnp.int32, L) + j0 for j0 in range(0, SPW, L)]

    def idx_b(b, carry):
        for k, j0 in enumerate(range(0, SPW, L)):
            sl = pl.ds(j0, L)
            idx_v[b, sl] = base + idx_v[b, sl] * SPW + jvecs[k]
        return carry

    lax.fori_loop(0, B, idx_b, 0)

    for v in range(V - 4, V):
        sh[v].wait()
    plsc.subcore_barrier()

    bufs = (gbuf0, gbuf1)
    gsems = (gs0, gs1)
    wsems = (ws0, ws1)

    def start_gather(c, buf, sem):
        return [pltpu.async_copy(cmb_sp.at[idx_v.at[c * G + i]],
                                 buf.at[i], sem)
                for i in range(G)]

    def start_write(c, buf, sem):
        return pltpu.async_copy(
            buf, out_hbm.at[pl.ds(c * G, G), pl.ds(s0, SPW)], sem)

    gh = [None] * NCHUNK
    wh = [None] * NCHUNK
    gh[0] = start_gather(0, bufs[0], gsems[0])
    for c in range(NCHUNK):
        p = c & 1
        if c >= 1:
            wh[c - 1].wait()
        if c + 1 < NCHUNK:
            gh[c + 1] = start_gather(c + 1, bufs[1 - p], gsems[1 - p])
        for h in gh[c]:
            h.wait()
        wh[c] = start_write(c, bufs[p], wsems[p])
    wh[NCHUNK - 1].wait()


@jax.jit
def kernel(x, emb_table, pos_encoding):
    xf = x.astype(jnp.int32).reshape(B * S)
    pos2d = pos_encoding.reshape(S, D).astype(jnp.float32)

    mesh = plsc.VectorSubcoreMesh(
        core_axis_name="c", subcore_axis_name="s",
        num_cores=NC, num_subcores=NS,
    )
    out = pl.kernel(
        _pe_body,
        out_type=jax.ShapeDtypeStruct((B, S, D), jnp.float32),
        mesh=mesh,
        scratch_types=[
            pltpu.VMEM_SHARED((NS * ROWS, D), jnp.float32),
            pltpu.VMEM((V, D), jnp.float32),
            pltpu.VMEM((SPW, D), jnp.float32),
            pltpu.VMEM((B, SPW), jnp.int32),
            pltpu.VMEM((G, SPW, D), jnp.float32),
            pltpu.VMEM((G, SPW, D), jnp.float32),
            pltpu.SemaphoreType.DMA,
            pltpu.SemaphoreType.DMA,
            pltpu.SemaphoreType.DMA,
            pltpu.SemaphoreType.DMA,
            pltpu.SemaphoreType.DMA,
        ],
    )(xf, emb_table, pos2d)
    return out


# speedup vs baseline: 1.0100x; 1.0100x over previous
import math

import jax
import jax.numpy as jnp
from jax import lax
from jax.experimental import pallas as pl
from jax.experimental.pallas import tpu as pltpu
from jax.experimental.pallas import tpu_sc as plsc

B = 64
S = 2048
D = 128
V = 9
L = 16
NC = 2
NS = 16
NW = NC * NS
SPW = S // NW
ROWS = V * SPW
SCALE = math.sqrt(D)
G = 2
NCHUNK = B // G


def _pe_body(x_hbm, tab_hbm, pos_hbm, out_hbm,
             cmb_sp, tabs_v, pos_v, idx_v, gbuf0, gbuf1,
             gs0, gs1, ws0, ws1, xs):
    cid = lax.axis_index("c")
    sid = lax.axis_index("s")
    wid = sid * NC + cid
    s0 = wid * SPW

    xh = [pltpu.async_copy(x_hbm.at[pl.ds(b * S + s0, SPW)], idx_v.at[b], xs)
          for b in range(B)]
    th = pltpu.async_copy(tab_hbm, tabs_v, gs0)
    ph = pltpu.async_copy(pos_hbm.at[pl.ds(s0, SPW)], pos_v, gs1)
    th.wait()
    ph.wait()

    for v in range(V):
        for d0 in range(0, D, L):
            sl = pl.ds(d0, L)
            tabs_v[v, sl] = tabs_v[v, sl] * SCALE

    pieces = (gbuf0.at[0], gbuf0.at[1], gbuf1.at[0], gbuf1.at[1])
    psems = (gs0, gs1, ws0, ws1)
    sh = [None] * V
    for v in range(V):
        p = v % 4
        if v >= 4:
            sh[v - 4].wait()
        piece = pieces[p]
        tv = [tabs_v[v, pl.ds(d0, L)] for d0 in range(0, D, L)]

        def build_j(j2, carry, piece=piece, tv=tv):
            for u in range(2):
                j = j2 * 2 + u
                for k, d0 in enumerate(range(0, D, L)):
                    sl = pl.ds(d0, L)
                    piece[j, sl] = pos_v[j, sl] + tv[k]
            return carry

        lax.fori_loop(0, SPW // 2, build_j, 0)
        sh[v] = pltpu.async_copy(
            piece, cmb_sp.at[pl.ds(sid * ROWS + v * SPW, SPW)], psems[p])

    for h in xh:
        h.wait()
    base = sid * ROWS
    jvecs = [lax.iota(jnp.int32, L) + j0 for j0 in range(0, SPW, L)]

    def idx_b(b, carry):
        for k, j0 in enumerate(range(0, SPW, L)):
            sl = pl.ds(j0, L)
            idx_v[b, sl] = base + idx_v[b, sl] * SPW + jvecs[k]
        return carry

    lax.fori_loop(0, B, idx_b, 0)

    for v in range(V - 4, V):
        sh[v].wait()
    plsc.subcore_barrier()

    bufs = (gbuf0, gbuf1)
    gsems = (gs0, gs1)
    wsems = (ws0, ws1)

    def start_gather(c, buf, sem):
        return [pltpu.async_copy(cmb_sp.at[idx_v.at[c * G + i]],
                                 buf.at[i], sem)
                for i in range(G)]

    def start_write(c, buf, sem):
        return pltpu.async_copy(
            buf, out_hbm.at[pl.ds(c * G, G), pl.ds(s0, SPW)], sem)

    gh = [None] * NCHUNK
    wh = [None] * NCHUNK
    gh[0] = start_gather(0, bufs[0], gsems[0])
    for c in range(NCHUNK):
        p = c & 1
        if c >= 1:
            wh[c - 1].wait()
        if c + 1 < NCHUNK:
            gh[c + 1] = start_gather(c + 1, bufs[1 - p], gsems[1 - p])
        for h in gh[c]:
            h.wait()
        wh[c] = start_write(c, bufs[p], wsems[p])
    wh[NCHUNK - 1].wait()


@jax.jit
def kernel(x, emb_table, pos_encoding):
    xf = x.astype(jnp.int32).reshape(B * S)
    pos2d = pos_encoding.reshape(S, D).astype(jnp.float32)

    mesh = plsc.VectorSubcoreMesh(
        core_axis_name="c", subcore_axis_name="s",
        num_cores=NC, num_subcores=NS,
    )
    out = pl.kernel(
        _pe_body,
        out_type=jax.ShapeDtypeStruct((B, S, D), jnp.float32),
        mesh=mesh,
        scratch_types=[
            pltpu.VMEM_SHARED((NS * ROWS, D), jnp.float32),
            pltpu.VMEM((V, D), jnp.float32),
            pltpu.VMEM((SPW, D), jnp.float32),
            pltpu.VMEM((B, SPW), jnp.int32),
            pltpu.VMEM((G, SPW, D), jnp.float32),
            pltpu.VMEM((G, SPW, D), jnp.float32),
            pltpu.SemaphoreType.DMA,
            pltpu.SemaphoreType.DMA,
            pltpu.SemaphoreType.DMA,
            pltpu.SemaphoreType.DMA,
            pltpu.SemaphoreType.DMA,
        ],
    )(xf, emb_table, pos2d)
    return out
